# baseline (speedup 1.0000x reference)
import functools
import jax
import jax.numpy as jnp
from jax import lax
from jax.experimental import pallas as pl
from jax.experimental.pallas import tpu as pltpu
from jax.experimental.pallas import tpu_sc as plsc

NC = 2
NS = 16
NW = NC * NS
LANES = 16
CH = 128
NSB = 8
UN = 4


def _vtake(v, idx):
    dn = lax.GatherDimensionNumbers(
        offset_dims=(), collapsed_slice_dims=(0,), start_index_map=(0,))
    return lax.gather(v, idx[:, None], dn, (1,),
                      mode=lax.GatherScatterMode.PROMISE_IN_BOUNDS)


def _lane_sum(v, lane):
    for r in (1, 2, 4, 8):
        v = v + _vtake(v, jnp.bitwise_xor(lane, r))
    return v


def _proj_body(x_ref, w1_ref, b1_ref, a_ref, b_ref, *, window):
    xb = x_ref[...]
    w = w1_ref[...]
    a_ref[...] = jnp.dot(xb, w[:window], preferred_element_type=jnp.float32)
    b_ref[...] = jnp.dot(xb, w[window:], preferred_element_type=jnp.float32) + b1_ref[...]


def _edge_logits_body(a_hbm, b_hbm, src_hbm, dst_hbm, w2_hbm, b2v_hbm,
                      e_out, denom_out,
                      sidx_blk, didx_blk, didx_cur, a_buf, b_buf, e_buf,
                      w2_v, b2_v, zeros_v, denom_sh, ga_sem, gb_sem,
                      *, hidden, stripe, nchunk):
    cid = lax.axis_index("c")
    sid = lax.axis_index("s")
    wid = sid * NC + cid
    lane = lax.iota(jnp.int32, 16)

    @pl.loop(0, stripe // LANES)
    def _zero(i):
        zeros_v[pl.ds(i * LANES, LANES)] = jnp.zeros((LANES,), jnp.float32)

    pltpu.sync_copy(zeros_v, denom_sh.at[pl.ds(sid * stripe, stripe)])
    pltpu.sync_copy(w2_hbm, w2_v)
    pltpu.sync_copy(b2v_hbm, b2_v)
    plsc.subcore_barrier()

    nc_h = hidden // LANES

    def compute(k):
        a_q, b_q = a_buf[k], b_buf[k]
        w2c = [w2_v[pl.ds(kk * LANES, LANES)] for kk in range(nc_h)]
        b2s = b2_v[...]

        @pl.loop(0, CH // LANES)
        def _grp(g):
            evec = jnp.zeros((LANES,), jnp.float32)
            for e16 in range(LANES):
                e = g * LANES + e16
                pa = jnp.zeros((LANES,), jnp.float32)
                for kk in range(nc_h):
                    va = a_q[e, pl.ds(kk * LANES, LANES)]
                    vb = b_q[e, pl.ds(kk * LANES, LANES)]
                    pa = pa + jnp.maximum(va + vb, 0.0) * w2c[kk]
                s = _lane_sum(pa, lane)
                evec = jnp.where(lane == e16, s, evec)
            e_buf[pl.ds(g * LANES, LANES)] = jnp.exp(evec + b2s)

    ph = nchunk // NSB
    wrow0 = wid * nchunk

    @pl.loop(0, NSB)
    def _sb(sb):
        row0 = wrow0 + sb * ph
        pltpu.sync_copy(src_hbm.at[pl.ds(row0, ph)], sidx_blk)
        pltpu.sync_copy(dst_hbm.at[pl.ds(row0, ph)], didx_blk)

        @pl.loop(0, ph // UN)
        def _quad(u):
            i0 = u * UN
            descs = []
            for k in range(UN):
                da = pltpu.async_copy(a_hbm.at[sidx_blk.at[i0 + k]],
                                      a_buf[k], ga_sem[k])
                db = pltpu.async_copy(b_hbm.at[didx_blk.at[i0 + k]],
                                      b_buf[k], gb_sem[k])
                descs.append((da, db))
            for k in range(UN):
                da, db = descs[k]
                da.wait()
                db.wait()
                compute(k)
                pltpu.sync_copy(e_buf, e_out.at[row0 + i0 + k])

                @pl.loop(0, CH // LANES)
                def _cpidx(g):
                    didx_cur[pl.ds(g * LANES, LANES)] = (
                        didx_blk[i0 + k, pl.ds(g * LANES, LANES)])

                pltpu.sync_copy(e_buf, denom_sh.at[didx_cur], add=True)

    plsc.subcore_barrier()
    npad = stripe * NS
    pltpu.sync_copy(denom_sh.at[pl.ds(sid * stripe, stripe)],
                    denom_out.at[pl.ds(cid * npad + sid * stripe, stripe)])


def _inv_body(d_ref, inv_ref, *, rows):
    inv_ref[...] = 1.0 / (d_ref[:rows] + d_ref[rows:] + 1e-16)


def _sum_body(d_ref, o_ref, *, rows):
    o_ref[...] = d_ref[:rows] + d_ref[rows:]


def _normalize_body(e_hbm, src_hbm, dst_hbm, inv_hbm, y_hbm,
                    alpha_out, yhat_out,
                    sidx, didx, e_buf, inv_buf, y_buf, al_buf, ct_buf, zeros_v,
                    inv_sh, y_sh, yhat_sh, sem_a, sem_b, *, stripe, nchunk):
    cid = lax.axis_index("c")
    sid = lax.axis_index("s")
    wid = sid * NC + cid
    sl = pl.ds(sid * stripe, stripe)

    @pl.loop(0, stripe // LANES)
    def _zero(i):
        zeros_v[pl.ds(i * LANES, LANES)] = jnp.zeros((LANES,), jnp.float32)

    pltpu.sync_copy(zeros_v, yhat_sh.at[sl])
    pltpu.sync_copy(inv_hbm.at[sl], inv_sh.at[sl])
    pltpu.sync_copy(y_hbm.at[sl], y_sh.at[sl])
    plsc.subcore_barrier()

    base0 = wid * (nchunk * CH)

    @pl.loop(0, nchunk)
    def _chunk(c):
        base = base0 + c * CH
        pltpu.sync_copy(src_hbm.at[pl.ds(base, CH)], sidx)
        pltpu.sync_copy(dst_hbm.at[pl.ds(base, CH)], didx)
        pltpu.sync_copy(e_hbm.at[pl.ds(base, CH)], e_buf)
        ci = pltpu.async_copy(inv_sh.at[didx], inv_buf, sem_a)
        cy = pltpu.async_copy(y_sh.at[sidx], y_buf, sem_b)
        ci.wait()
        cy.wait()

        @pl.loop(0, CH // LANES)
        def _grp(g):
            s = pl.ds(g * LANES, LANES)
            al = e_buf[s] * inv_buf[s]
            al_buf[s] = al
            ct_buf[s] = al * y_buf[s]

        pltpu.sync_copy(al_buf, alpha_out.at[pl.ds(base, CH)])
        pltpu.sync_copy(ct_buf, yhat_sh.at[didx], add=True)

    plsc.subcore_barrier()
    pltpu.sync_copy(yhat_sh.at[sl],
                    yhat_out.at[pl.ds(cid * (stripe * NS) + sid * stripe, stripe)])


def kernel(x, y, edge_index, batch, W1, b1, W2, b2):
    n_nodes, window = x.shape
    hidden = W1.shape[1]
    n_edges = edge_index.shape[1]
    f32 = jnp.float32

    npad = ((n_nodes + 1 + 127) // 128) * 128
    rows = npad // 128
    stripe = npad // NS
    unit = NW * CH * NSB * UN
    e_pad = ((n_edges + unit - 1) // unit) * unit
    nchunk = e_pad // (NW * CH)

    src = edge_index[0].astype(jnp.int32)
    dst = edge_index[1].astype(jnp.int32)
    pad_idx = jnp.full((e_pad - n_edges,), n_nodes, jnp.int32)
    srcp = jnp.concatenate([src, pad_idx])
    dstp = jnp.concatenate([dst, pad_idx])
    x_p = jnp.zeros((npad, window), f32).at[:n_nodes].set(x)
    y_p = jnp.zeros((npad,), f32).at[:n_nodes].set(y)
    w2f = W2.reshape(hidden)
    b2v = jnp.broadcast_to(b2.reshape(1), (LANES,))

    blk = 256
    a_mat, b_mat = pl.pallas_call(
        functools.partial(_proj_body, window=window),
        grid=(npad // blk,),
        in_specs=[
            pl.BlockSpec((blk, window), lambda i: (i, 0)),
            pl.BlockSpec((2 * window, hidden), lambda i: (0, 0)),
            pl.BlockSpec((1, hidden), lambda i: (0, 0)),
        ],
        out_specs=[
            pl.BlockSpec((blk, hidden), lambda i: (i, 0)),
            pl.BlockSpec((blk, hidden), lambda i: (i, 0)),
        ],
        out_shape=[
            jax.ShapeDtypeStruct((npad, hidden), f32),
            jax.ShapeDtypeStruct((npad, hidden), f32),
        ],
    )(x_p, W1, b1.reshape(1, hidden))

    mesh = plsc.VectorSubcoreMesh(core_axis_name="c", subcore_axis_name="s")
    e_arr, denom_p = pl.kernel(
        functools.partial(_edge_logits_body, hidden=hidden, stripe=stripe,
                          nchunk=nchunk),
        out_type=[
            jax.ShapeDtypeStruct((e_pad // CH, CH), f32),
            jax.ShapeDtypeStruct((NC * npad,), f32),
        ],
        mesh=mesh,
        compiler_params=pltpu.CompilerParams(use_tc_tiling_on_sc=False),
        scratch_types=[
            pltpu.VMEM((nchunk // NSB, CH), jnp.int32),
            pltpu.VMEM((nchunk // NSB, CH), jnp.int32),
            pltpu.VMEM((CH,), jnp.int32),
            [pltpu.VMEM((CH, hidden), f32) for _ in range(UN)],
            [pltpu.VMEM((CH, hidden), f32) for _ in range(UN)],
            pltpu.VMEM((CH,), f32),
            pltpu.VMEM((hidden,), f32),
            pltpu.VMEM((LANES,), f32),
            pltpu.VMEM((stripe,), f32),
            pltpu.VMEM_SHARED((npad,), f32),
            [pltpu.SemaphoreType.DMA for _ in range(UN)],
            [pltpu.SemaphoreType.DMA for _ in range(UN)],
        ],
    )(a_mat, b_mat, srcp.reshape(-1, CH), dstp.reshape(-1, CH), w2f, b2v)

    inv = pl.pallas_call(
        functools.partial(_inv_body, rows=rows),
        out_shape=jax.ShapeDtypeStruct((rows, 128), f32),
    )(denom_p.reshape(2 * rows, 128)).reshape(npad)

    alpha_full, yhat_p = pl.kernel(
        functools.partial(_normalize_body, stripe=stripe, nchunk=nchunk),
        out_type=[
            jax.ShapeDtypeStruct((e_pad,), f32),
            jax.ShapeDtypeStruct((NC * npad,), f32),
        ],
        mesh=mesh,
        compiler_params=pltpu.CompilerParams(use_tc_tiling_on_sc=False),
        scratch_types=[
            pltpu.VMEM((CH,), jnp.int32),
            pltpu.VMEM((CH,), jnp.int32),
            pltpu.VMEM((CH,), f32),
            pltpu.VMEM((CH,), f32),
            pltpu.VMEM((CH,), f32),
            pltpu.VMEM((CH,), f32),
            pltpu.VMEM((CH,), f32),
            pltpu.VMEM((stripe,), f32),
            pltpu.VMEM_SHARED((npad,), f32),
            pltpu.VMEM_SHARED((npad,), f32),
            pltpu.VMEM_SHARED((npad,), f32),
            pltpu.SemaphoreType.DMA,
            pltpu.SemaphoreType.DMA,
        ],
    )(e_arr.reshape(e_pad), srcp, dstp, inv, y_p)

    yhat = pl.pallas_call(
        functools.partial(_sum_body, rows=rows),
        out_shape=jax.ShapeDtypeStruct((rows, 128), f32),
    )(yhat_p.reshape(2 * rows, 128)).reshape(npad)

    y_hat = yhat[:n_nodes]
    alpha = alpha_full[:n_edges].reshape(n_edges, 1)
    return (y_hat, lax.stop_gradient(alpha))


# speedup vs baseline: 1.3418x; 1.3418x over previous
import functools
import jax
import jax.numpy as jnp
from jax import lax
from jax.experimental import pallas as pl
from jax.experimental.pallas import tpu as pltpu
from jax.experimental.pallas import tpu_sc as plsc

NC = 2
NS = 16
NW = NC * NS
LANES = 16
CH = 128
NSB = 8
UN = 4


def _vtake(v, idx):
    dn = lax.GatherDimensionNumbers(
        offset_dims=(), collapsed_slice_dims=(0,), start_index_map=(0,))
    return lax.gather(v, idx[:, None], dn, (1,),
                      mode=lax.GatherScatterMode.PROMISE_IN_BOUNDS)


def _lane_sum(v, lane):
    for r in (1, 2, 4, 8):
        v = v + _vtake(v, jnp.bitwise_xor(lane, r))
    return v


def _proj_body(x_ref, w1_ref, b1_ref, a_ref, b_ref, *, window):
    xb = x_ref[...]
    w = w1_ref[...]
    a_ref[...] = jnp.dot(xb, w[:window], preferred_element_type=jnp.float32)
    b_ref[...] = jnp.dot(xb, w[window:], preferred_element_type=jnp.float32) + b1_ref[...]


def _edge_logits_body(a_hbm, b_hbm, src_hbm, dst_hbm, w2_hbm, b2v_hbm,
                      e_out, denom_out,
                      sidx_blk, didx_blk, didx_cur, a_buf, b_buf, e_buf,
                      w2_v, b2_v, zeros_v, denom_sh, ga_sem, gb_sem,
                      w_sem, s_sem, *, hidden, stripe, nchunk):
    cid = lax.axis_index("c")
    sid = lax.axis_index("s")
    wid = sid * NC + cid
    lane = lax.iota(jnp.int32, 16)

    @pl.loop(0, stripe // LANES)
    def _zero(i):
        zeros_v[pl.ds(i * LANES, LANES)] = jnp.zeros((LANES,), jnp.float32)

    pltpu.sync_copy(zeros_v, denom_sh.at[pl.ds(sid * stripe, stripe)])
    pltpu.sync_copy(w2_hbm, w2_v)
    pltpu.sync_copy(b2v_hbm, b2_v)
    plsc.subcore_barrier()

    nc_h = hidden // LANES

    def compute(k):
        a_q, b_q, e_q = a_buf[k], b_buf[k], e_buf[k]
        w2c = [w2_v[pl.ds(kk * LANES, LANES)] for kk in range(nc_h)]
        b2s = b2_v[...]

        @pl.loop(0, CH // LANES)
        def _grp(g):
            evec = jnp.zeros((LANES,), jnp.float32)
            for e16 in range(LANES):
                e = g * LANES + e16
                pa = jnp.zeros((LANES,), jnp.float32)
                for kk in range(nc_h):
                    va = a_q[e, pl.ds(kk * LANES, LANES)]
                    vb = b_q[e, pl.ds(kk * LANES, LANES)]
                    pa = pa + jnp.maximum(va + vb, 0.0) * w2c[kk]
                s = _lane_sum(pa, lane)
                evec = jnp.where(lane == e16, s, evec)
            e_q[pl.ds(g * LANES, LANES)] = jnp.exp(evec + b2s)

    ph = nchunk // NSB
    wrow0 = wid * nchunk

    def quad(u, row0, drain):
        i0 = u * UN
        descs = []
        for k in range(UN):
            da = pltpu.async_copy(a_hbm.at[sidx_blk.at[i0 + k]],
                                  a_buf[k], ga_sem[k])
            db = pltpu.async_copy(b_hbm.at[didx_blk.at[i0 + k]],
                                  b_buf[k], gb_sem[k])
            descs.append((da, db))
        for k in range(UN):
            da, db = descs[k]
            da.wait()
            db.wait()
            if drain:
                drain_ws(k)

            @pl.loop(0, CH // LANES)
            def _cpidx(g):
                didx_cur[k][pl.ds(g * LANES, LANES)] = (
                    didx_blk[i0 + k, pl.ds(g * LANES, LANES)])

            compute(k)
            pltpu.async_copy(e_buf[k], e_out.at[row0 + i0 + k], w_sem[k])
            pltpu.async_copy(e_buf[k], denom_sh.at[didx_cur[k]], s_sem[k],
                             add=True)

    def drain_ws(k):
        pltpu.make_async_copy(e_buf[k], e_out.at[0], w_sem[k]).wait()
        pltpu.make_async_copy(e_buf[k], denom_sh.at[didx_cur[k]],
                              s_sem[k]).wait()

    @pl.loop(0, NSB)
    def _sb(sb):
        row0 = wrow0 + sb * ph
        pltpu.sync_copy(src_hbm.at[pl.ds(row0, ph)], sidx_blk)
        pltpu.sync_copy(dst_hbm.at[pl.ds(row0, ph)], didx_blk)
        quad(0, row0, False)

        @pl.loop(1, ph // UN)
        def _quad(u):
            quad(u, row0, True)

        for k in range(UN):
            drain_ws(k)

    plsc.subcore_barrier()
    npad = stripe * NS
    pltpu.sync_copy(denom_sh.at[pl.ds(sid * stripe, stripe)],
                    denom_out.at[pl.ds(cid * npad + sid * stripe, stripe)])


def _inv_body(d_ref, inv_ref, *, rows):
    inv_ref[...] = 1.0 / (d_ref[:rows] + d_ref[rows:] + 1e-16)


def _sum_body(d_ref, o_ref, *, rows):
    o_ref[...] = d_ref[:rows] + d_ref[rows:]


def _normalize_body(e_hbm, src_hbm, dst_hbm, inv_hbm, y_hbm,
                    alpha_out, yhat_out,
                    sidx_blk, didx_blk, e_blk, al_blk, didx_cur, inv_buf,
                    y_buf, ct_buf, zeros_v, inv_sh, y_sh, yhat_sh,
                    gi_sem, gy_sem, s_sem, *, stripe, nchunk):
    cid = lax.axis_index("c")
    sid = lax.axis_index("s")
    wid = sid * NC + cid
    sl = pl.ds(sid * stripe, stripe)

    @pl.loop(0, stripe // LANES)
    def _zero(i):
        zeros_v[pl.ds(i * LANES, LANES)] = jnp.zeros((LANES,), jnp.float32)

    pltpu.sync_copy(zeros_v, yhat_sh.at[sl])
    pltpu.sync_copy(inv_hbm.at[sl], inv_sh.at[sl])
    pltpu.sync_copy(y_hbm.at[sl], y_sh.at[sl])
    plsc.subcore_barrier()

    ph = nchunk // NSB
    wrow0 = wid * nchunk

    def drain_s(k):
        pltpu.make_async_copy(ct_buf[k], yhat_sh.at[didx_cur[k]],
                              s_sem[k]).wait()

    def quad(u, drain):
        i0 = u * UN
        descs = []
        for k in range(UN):
            di = pltpu.async_copy(inv_sh.at[didx_blk.at[i0 + k]],
                                  inv_buf[k], gi_sem[k])
            dy = pltpu.async_copy(y_sh.at[sidx_blk.at[i0 + k]],
                                  y_buf[k], gy_sem[k])
            descs.append((di, dy))
        for k in range(UN):
            di, dy = descs[k]
            di.wait()
            dy.wait()
            if drain:
                drain_s(k)

            @pl.loop(0, CH // LANES)
            def _grp(g):
                s = pl.ds(g * LANES, LANES)
                didx_cur[k][s] = didx_blk[i0 + k, s]
                al = e_blk[i0 + k, s] * inv_buf[k][s]
                al_blk[i0 + k, s] = al
                ct_buf[k][s] = al * y_buf[k][s]

            pltpu.async_copy(ct_buf[k], yhat_sh.at[didx_cur[k]], s_sem[k],
                             add=True)

    @pl.loop(0, NSB)
    def _sb(sb):
        row0 = wrow0 + sb * ph
        pltpu.sync_copy(src_hbm.at[pl.ds(row0, ph)], sidx_blk)
        pltpu.sync_copy(dst_hbm.at[pl.ds(row0, ph)], didx_blk)
        pltpu.sync_copy(e_hbm.at[pl.ds(row0, ph)], e_blk)
        quad(0, False)

        @pl.loop(1, ph // UN)
        def _quad(u):
            quad(u, True)

        for k in range(UN):
            drain_s(k)
        pltpu.sync_copy(al_blk, alpha_out.at[pl.ds(row0, ph)])

    plsc.subcore_barrier()
    pltpu.sync_copy(yhat_sh.at[sl],
                    yhat_out.at[pl.ds(cid * (stripe * NS) + sid * stripe, stripe)])


def kernel(x, y, edge_index, batch, W1, b1, W2, b2):
    n_nodes, window = x.shape
    hidden = W1.shape[1]
    n_edges = edge_index.shape[1]
    f32 = jnp.float32

    npad = ((n_nodes + 1 + 127) // 128) * 128
    rows = npad // 128
    stripe = npad // NS
    unit = NW * CH * NSB * UN
    e_pad = ((n_edges + unit - 1) // unit) * unit
    nchunk = e_pad // (NW * CH)

    src = edge_index[0].astype(jnp.int32)
    dst = edge_index[1].astype(jnp.int32)
    pad_idx = jnp.full((e_pad - n_edges,), n_nodes, jnp.int32)
    srcp = jnp.concatenate([src, pad_idx])
    dstp = jnp.concatenate([dst, pad_idx])
    x_p = jnp.zeros((npad, window), f32).at[:n_nodes].set(x)
    y_p = jnp.zeros((npad,), f32).at[:n_nodes].set(y)
    w2f = W2.reshape(hidden)
    b2v = jnp.broadcast_to(b2.reshape(1), (LANES,))

    blk = 256
    a_mat, b_mat = pl.pallas_call(
        functools.partial(_proj_body, window=window),
        grid=(npad // blk,),
        in_specs=[
            pl.BlockSpec((blk, window), lambda i: (i, 0)),
            pl.BlockSpec((2 * window, hidden), lambda i: (0, 0)),
            pl.BlockSpec((1, hidden), lambda i: (0, 0)),
        ],
        out_specs=[
            pl.BlockSpec((blk, hidden), lambda i: (i, 0)),
            pl.BlockSpec((blk, hidden), lambda i: (i, 0)),
        ],
        out_shape=[
            jax.ShapeDtypeStruct((npad, hidden), f32),
            jax.ShapeDtypeStruct((npad, hidden), f32),
        ],
    )(x_p, W1, b1.reshape(1, hidden))

    mesh = plsc.VectorSubcoreMesh(core_axis_name="c", subcore_axis_name="s")
    e_arr, denom_p = pl.kernel(
        functools.partial(_edge_logits_body, hidden=hidden, stripe=stripe,
                          nchunk=nchunk),
        out_type=[
            jax.ShapeDtypeStruct((e_pad // CH, CH), f32),
            jax.ShapeDtypeStruct((NC * npad,), f32),
        ],
        mesh=mesh,
        compiler_params=pltpu.CompilerParams(use_tc_tiling_on_sc=False),
        scratch_types=[
            pltpu.VMEM((nchunk // NSB, CH), jnp.int32),
            pltpu.VMEM((nchunk // NSB, CH), jnp.int32),
            [pltpu.VMEM((CH,), jnp.int32) for _ in range(UN)],
            [pltpu.VMEM((CH, hidden), f32) for _ in range(UN)],
            [pltpu.VMEM((CH, hidden), f32) for _ in range(UN)],
            [pltpu.VMEM((CH,), f32) for _ in range(UN)],
            pltpu.VMEM((hidden,), f32),
            pltpu.VMEM((LANES,), f32),
            pltpu.VMEM((stripe,), f32),
            pltpu.VMEM_SHARED((npad,), f32),
            [pltpu.SemaphoreType.DMA for _ in range(UN)],
            [pltpu.SemaphoreType.DMA for _ in range(UN)],
            [pltpu.SemaphoreType.DMA for _ in range(UN)],
            [pltpu.SemaphoreType.DMA for _ in range(UN)],
        ],
    )(a_mat, b_mat, srcp.reshape(-1, CH), dstp.reshape(-1, CH), w2f, b2v)

    inv = pl.pallas_call(
        functools.partial(_inv_body, rows=rows),
        out_shape=jax.ShapeDtypeStruct((rows, 128), f32),
    )(denom_p.reshape(2 * rows, 128)).reshape(npad)

    alpha_full, yhat_p = pl.kernel(
        functools.partial(_normalize_body, stripe=stripe, nchunk=nchunk),
        out_type=[
            jax.ShapeDtypeStruct((e_pad // CH, CH), f32),
            jax.ShapeDtypeStruct((NC * npad,), f32),
        ],
        mesh=mesh,
        compiler_params=pltpu.CompilerParams(use_tc_tiling_on_sc=False),
        scratch_types=[
            pltpu.VMEM((nchunk // NSB, CH), jnp.int32),
            pltpu.VMEM((nchunk // NSB, CH), jnp.int32),
            pltpu.VMEM((nchunk // NSB, CH), f32),
            pltpu.VMEM((nchunk // NSB, CH), f32),
            [pltpu.VMEM((CH,), jnp.int32) for _ in range(UN)],
            [pltpu.VMEM((CH,), f32) for _ in range(UN)],
            [pltpu.VMEM((CH,), f32) for _ in range(UN)],
            [pltpu.VMEM((CH,), f32) for _ in range(UN)],
            pltpu.VMEM((stripe,), f32),
            pltpu.VMEM_SHARED((npad,), f32),
            pltpu.VMEM_SHARED((npad,), f32),
            pltpu.VMEM_SHARED((npad,), f32),
            [pltpu.SemaphoreType.DMA for _ in range(UN)],
            [pltpu.SemaphoreType.DMA for _ in range(UN)],
            [pltpu.SemaphoreType.DMA for _ in range(UN)],
        ],
    )(e_arr, srcp.reshape(-1, CH), dstp.reshape(-1, CH), inv, y_p)

    yhat = pl.pallas_call(
        functools.partial(_sum_body, rows=rows),
        out_shape=jax.ShapeDtypeStruct((rows, 128), f32),
    )(yhat_p.reshape(2 * rows, 128)).reshape(npad)

    y_hat = yhat[:n_nodes]
    alpha = alpha_full.reshape(e_pad)[:n_edges].reshape(n_edges, 1)
    return (y_hat, lax.stop_gradient(alpha))
